# baseline (speedup 1.0000x reference)
import functools

import jax
import jax.numpy as jnp
from jax.experimental import pallas as pl
from jax.experimental.pallas import tpu as pltpu

_BM = 200


def _fused_body(s0_ref, s1_ref, h_ref, w_ref, b_ref, o0_ref, o1_ref, hw_ref):
    @pl.when(pl.program_id(0) == 0)
    def _project():
        hw_ref[...] = jnp.dot(
            h_ref[...], w_ref[...], preferred_element_type=jnp.float32
        )

    hw = hw_ref[...]
    b = b_ref[...]
    o0_ref[...] = (
        jnp.dot(s0_ref[...], hw, preferred_element_type=jnp.float32) + b
    )
    o1_ref[...] = (
        jnp.dot(s1_ref[...], hw, preferred_element_type=jnp.float32) + b
    )


@jax.jit
def kernel(structure, H, W, bias):
    n, in_f = H.shape
    out_f = W.shape[1]
    bias2d = bias.reshape(1, out_f)
    half_blocks = (n // 2) // _BM
    out0, out1 = pl.pallas_call(
        _fused_body,
        grid=(half_blocks,),
        in_specs=[
            pl.BlockSpec((_BM, n), lambda i: (i, 0)),
            pl.BlockSpec((_BM, n), lambda i, hb=half_blocks: (i + hb, 0)),
            pl.BlockSpec((n, in_f), lambda i: (0, 0)),
            pl.BlockSpec((in_f, out_f), lambda i: (0, 0)),
            pl.BlockSpec((1, out_f), lambda i: (0, 0)),
        ],
        out_specs=[
            pl.BlockSpec((_BM, out_f), lambda i: (i, 0)),
            pl.BlockSpec((_BM, out_f), lambda i: (i, 0)),
        ],
        out_shape=[
            jax.ShapeDtypeStruct((n // 2, out_f), jnp.float32),
            jax.ShapeDtypeStruct((n // 2, out_f), jnp.float32),
        ],
        scratch_shapes=[pltpu.VMEM((n, out_f), jnp.float32)],
    )(structure, structure, H, W, bias2d)
    return jnp.concatenate([out0, out1], axis=0)


# speedup vs baseline: 1.0378x; 1.0378x over previous
import functools

import jax
import jax.numpy as jnp
from jax.experimental import pallas as pl
from jax.experimental.pallas import tpu as pltpu

_BM = 400


def _fused_body(s_ref, h_ref, w_ref, b_ref, out_ref, hw_ref):
    @pl.when(pl.program_id(0) == 0)
    def _project():
        hw_ref[...] = jnp.dot(
            h_ref[...], w_ref[...], preferred_element_type=jnp.float32
        )

    out_ref[...] = (
        jnp.dot(s_ref[...], hw_ref[...], preferred_element_type=jnp.float32)
        + b_ref[...]
    )


@jax.jit
def kernel(structure, H, W, bias):
    n, in_f = H.shape
    out_f = W.shape[1]
    bias2d = bias.reshape(1, out_f)
    grid = (structure.shape[0] // _BM,)
    return pl.pallas_call(
        _fused_body,
        grid=grid,
        in_specs=[
            pl.BlockSpec((_BM, n), lambda i: (i, 0)),
            pl.BlockSpec((n, in_f), lambda i: (0, 0)),
            pl.BlockSpec((in_f, out_f), lambda i: (0, 0)),
            pl.BlockSpec((1, out_f), lambda i: (0, 0)),
        ],
        out_specs=pl.BlockSpec((_BM, out_f), lambda i: (i, 0)),
        out_shape=jax.ShapeDtypeStruct((structure.shape[0], out_f), jnp.float32),
        scratch_shapes=[pltpu.VMEM((n, out_f), jnp.float32)],
    )(structure, H, W, bias2d)
